# Initial kernel scaffold; baseline (speedup 1.0000x reference)
import jax
import jax.numpy as jnp
from jax.experimental import pallas as pl


def kernel(x, edge_index, ffn_w1, ffn_b1, bn1_g, bn1_b, ffn_w2, ffn_b2, bn2_g, bn2_b, w1, b1, w2, b2, w3, b3, w4, b4, w5, b5, fc_w, fc_b):
    raise NotImplementedError("write your pallas kernel here")


# speedup vs baseline: 39.8387x; 39.8387x over previous
import functools
import math

import jax
import jax.numpy as jnp
from jax import lax
from jax.experimental import pallas as pl
from jax.experimental.pallas import tpu as pltpu
from jax.experimental.pallas import tpu_sc as plsc

_N = 99900
_E = 3196800
_NTILES = 32
_LANES = 16
_GRP = 128
_K = 8
_CHUNK = _K * _GRP
_EPT = _E // _NTILES
_NCHUNK = -(-_EPT // _CHUNK)
_EPT_PAD = _NCHUNK * _CHUNK
_ROWTILE = _EPT_PAD // _GRP
_NPAD = 100352
_BLK = 1024
_GRID = _NPAD // _BLK
_BN_SCALE = 1.0 / math.sqrt(1.0 + 1e-5)


def _fill_rows(buf, nrows, val):
    def body(i, c):
        buf[i, :] = jnp.full((_LANES,), val, jnp.float32)
        return c
    lax.fori_loop(0, nrows, body, 0)


def _zero_my_accum_slice(rows, accum, sid, copyrows):
    lo = sid * copyrows
    n_full = copyrows // _CHUNK
    rem = copyrows % _CHUNK
    for z in range(n_full):
        pltpu.sync_copy(rows, accum.at[pl.ds(lo + z * _CHUNK, _CHUNK)])
    if rem:
        pltpu.sync_copy(rows.at[pl.ds(0, rem)],
                        accum.at[pl.ds(lo + n_full * _CHUNK, rem)])


def _make_sc_agg(npad, rowtile, nchunk, interpret=False):
    copyrows = npad // 16

    def body(g_hbm, src_hbm, dst_hbm, out_hbm, srcb, dstb, rows, accum, sem):
        cid = lax.axis_index("c")
        sid = lax.axis_index("s")
        wid = cid * 16 + sid
        _fill_rows(rows, _CHUNK, 0.0)
        _zero_my_accum_slice(rows, accum, sid, copyrows)
        plsc.subcore_barrier()
        base = wid * rowtile

        def chunk(m, c):
            r0 = base + m * _K
            pltpu.sync_copy(src_hbm.at[pl.ds(r0, _K)], srcb)
            pltpu.sync_copy(dst_hbm.at[pl.ds(r0, _K)], dstb)
            descs = [
                pltpu.async_copy(g_hbm.at[srcb.at[j]],
                                 rows.at[pl.ds(j * _GRP, _GRP)], sem)
                for j in range(_K)
            ]
            for d in descs:
                d.wait()
            for j in range(_K):
                pltpu.sync_copy(rows.at[pl.ds(j * _GRP, _GRP)],
                                accum.at[dstb.at[j]], add=True)
            return c

        lax.fori_loop(0, nchunk, chunk, 0)
        plsc.subcore_barrier()
        lo = sid * copyrows
        pltpu.sync_copy(accum.at[pl.ds(lo, copyrows)],
                        out_hbm.at[cid].at[pl.ds(lo, copyrows)])

    return pl.kernel(
        body,
        out_type=jax.ShapeDtypeStruct((2, npad, _LANES), jnp.float32),
        mesh=plsc.VectorSubcoreMesh(core_axis_name="c", subcore_axis_name="s"),
        scratch_types=[
            pltpu.VMEM((_K, _GRP), jnp.int32),
            pltpu.VMEM((_K, _GRP), jnp.int32),
            pltpu.VMEM((_CHUNK, _LANES), jnp.float32),
            pltpu.VMEM_SHARED((npad, _LANES), jnp.float32),
            pltpu.SemaphoreType.DMA,
        ],
        compiler_params=pltpu.CompilerParams(use_tc_tiling_on_sc=False),
        interpret=interpret,
    )


def _make_sc_deg(npad, rowtile, nchunk, interpret=False):
    copyrows = npad // 16

    def body(dst_hbm, out_hbm, dstb, rows, accum):
        cid = lax.axis_index("c")
        sid = lax.axis_index("s")
        wid = cid * 16 + sid
        _fill_rows(rows, _CHUNK, 0.0)
        _zero_my_accum_slice(rows, accum, sid, copyrows)
        _fill_rows(rows, _CHUNK, 1.0)
        plsc.subcore_barrier()
        base = wid * rowtile

        def chunk(m, c):
            r0 = base + m * _K
            pltpu.sync_copy(dst_hbm.at[pl.ds(r0, _K)], dstb)
            for j in range(_K):
                pltpu.sync_copy(rows.at[pl.ds(j * _GRP, _GRP)],
                                accum.at[dstb.at[j]], add=True)
            return c

        lax.fori_loop(0, nchunk, chunk, 0)
        plsc.subcore_barrier()
        lo = sid * copyrows
        pltpu.sync_copy(accum.at[pl.ds(lo, copyrows)],
                        out_hbm.at[cid].at[pl.ds(lo, copyrows)])

    return pl.kernel(
        body,
        out_type=jax.ShapeDtypeStruct((2, npad, _LANES), jnp.float32),
        mesh=plsc.VectorSubcoreMesh(core_axis_name="c", subcore_axis_name="s"),
        scratch_types=[
            pltpu.VMEM((_K, _GRP), jnp.int32),
            pltpu.VMEM((_CHUNK, _LANES), jnp.float32),
            pltpu.VMEM_SHARED((npad, _LANES), jnp.float32),
        ],
        compiler_params=pltpu.CompilerParams(use_tc_tiling_on_sc=False),
        interpret=interpret,
    )


def _row_spec(d):
    return pl.BlockSpec((_BLK, d), lambda i: (i, 0))


def _full_spec(shape):
    nd = len(shape)
    return pl.BlockSpec(shape, lambda i, _n=nd: (0,) * _n)


def _part_spec():
    return pl.BlockSpec((2, _BLK, _LANES), lambda i: (0, i, 0))


def _f0_body(p_ref, x_ref, fw1, fb1, g1g, g1b, fw2, fb2, g2g, g2b, w1_ref,
             dinv_ref, ga_ref, gb_ref):
    cnt = p_ref[0, :, 0:1] + p_ref[1, :, 0:1]
    dinv = lax.rsqrt(1.0 + cnt)
    h = x_ref[...] @ fw1[...] + fb1[...]
    h = h * (g1g[...] * _BN_SCALE) + g1b[...]
    h = jnp.maximum(h, 0.0)
    h = h @ fw2[...] + fb2[...]
    h = h * (g2g[...] * _BN_SCALE) + g2b[...]
    g = dinv * (h @ w1_ref[...])
    dinv_ref[...] = dinv
    ga_ref[...] = g[:, :16]
    gb_ref[...] = jnp.concatenate(
        [g[:, 16:], jnp.zeros((_BLK, 7), jnp.float32)], axis=1)


_f0_call = pl.pallas_call(
    _f0_body,
    grid=(_GRID,),
    in_specs=[
        _part_spec(),
        _row_spec(25),
        _full_spec((25, 100)), _full_spec((1, 100)),
        _full_spec((1, 100)), _full_spec((1, 100)),
        _full_spec((100, 25)), _full_spec((1, 25)),
        _full_spec((1, 25)), _full_spec((1, 25)),
        _full_spec((25, 25)),
    ],
    out_specs=[_row_spec(1), _row_spec(16), _row_spec(16)],
    out_shape=[
        jax.ShapeDtypeStruct((_NPAD, 1), jnp.float32),
        jax.ShapeDtypeStruct((_NPAD, 16), jnp.float32),
        jax.ShapeDtypeStruct((_NPAD, 16), jnp.float32),
    ],
)


def _f1_body(sa_ref, sb_ref, ga_ref, gb_ref, dinv_ref, b_ref, w_ref, out_ref):
    agg_a = sa_ref[0] + sa_ref[1] + ga_ref[...]
    agg_b = sb_ref[0] + sb_ref[1] + gb_ref[...]
    agg = jnp.concatenate([agg_a, agg_b[:, :9]], axis=1)
    dinv = dinv_ref[...]
    h = jnp.maximum(dinv * agg + b_ref[...], 0.0)
    out_ref[...] = dinv * (h @ w_ref[...])


_f1_call = pl.pallas_call(
    _f1_body,
    grid=(_GRID,),
    in_specs=[
        _part_spec(), _part_spec(), _row_spec(16), _row_spec(16),
        _row_spec(1), _full_spec((1, 25)), _full_spec((25, 16)),
    ],
    out_specs=_row_spec(16),
    out_shape=jax.ShapeDtypeStruct((_NPAD, 16), jnp.float32),
)


def _make_mid(d, d2):
    def body(s_ref, g_ref, dinv_ref, b_ref, w_ref, out_ref):
        agg = (s_ref[0] + s_ref[1] + g_ref[...])[:, :d]
        dinv = dinv_ref[...]
        h = jnp.maximum(dinv * agg + b_ref[...], 0.0)
        g2 = dinv * (h @ w_ref[...])
        if d2 < 16:
            g2 = jnp.concatenate(
                [g2, jnp.zeros((_BLK, 16 - d2), jnp.float32)], axis=1)
        out_ref[...] = g2

    return pl.pallas_call(
        body,
        grid=(_GRID,),
        in_specs=[
            _part_spec(), _row_spec(16), _row_spec(1),
            _full_spec((1, d)), _full_spec((d, d2)),
        ],
        out_specs=_row_spec(16),
        out_shape=jax.ShapeDtypeStruct((_NPAD, 16), jnp.float32),
    )


_f2_call = _make_mid(16, 16)
_f3_call = _make_mid(16, 8)
_f4_call = _make_mid(8, 4)


def _flast_body(s_ref, g_ref, dinv_ref, b_ref, out_ref):
    agg = (s_ref[0] + s_ref[1] + g_ref[...])[:, :4]
    out_ref[...] = jnp.maximum(dinv_ref[...] * agg + b_ref[...], 0.0)


_flast_call = pl.pallas_call(
    _flast_body,
    grid=(_GRID,),
    in_specs=[_part_spec(), _row_spec(16), _row_spec(1), _full_spec((1, 4))],
    out_specs=_row_spec(4),
    out_shape=jax.ShapeDtypeStruct((_NPAD, 4), jnp.float32),
)


def _f6_body(h_ref, w_ref, b_ref, out_ref):
    out_ref[...] = h_ref[...] @ w_ref[...] + b_ref[...]


_f6_call = pl.pallas_call(
    _f6_body,
    grid=(1,),
    in_specs=[_full_spec((333, 1200)), _full_spec((1200, 4)),
              _full_spec((1, 4))],
    out_specs=_full_spec((333, 4)),
    out_shape=jax.ShapeDtypeStruct((333, 4), jnp.float32),
)


_sc_agg = _make_sc_agg(_NPAD, _ROWTILE, _NCHUNK)
_sc_deg = _make_sc_deg(_NPAD, _ROWTILE, _NCHUNK)


def _row(v):
    return v.reshape(1, -1)


def kernel(x, edge_index, ffn_w1, ffn_b1, bn1_g, bn1_b, ffn_w2, ffn_b2,
           bn2_g, bn2_b, w1, b1, w2, b2, w3, b3, w4, b4, w5, b5, fc_w, fc_b):
    f32 = jnp.float32
    x = x.astype(f32)
    src = edge_index[0]
    dst = edge_index[1]
    pad = _EPT_PAD - _EPT
    srcp = jnp.pad(src.reshape(_NTILES, _EPT), ((0, 0), (0, pad)),
                   constant_values=_N).reshape(-1, _GRP)
    dstp = jnp.pad(dst.reshape(_NTILES, _EPT), ((0, 0), (0, pad)),
                   constant_values=_N).reshape(-1, _GRP)
    xp = jnp.pad(x, ((0, _NPAD - _N), (0, 0)))

    p_deg = _sc_deg(dstp)
    dinv, g1a, g1b = _f0_call(
        p_deg, xp, ffn_w1, _row(ffn_b1), _row(bn1_g), _row(bn1_b),
        ffn_w2, _row(ffn_b2), _row(bn2_g), _row(bn2_b), w1)
    sa = _sc_agg(g1a, srcp, dstp)
    sb = _sc_agg(g1b, srcp, dstp)
    g2 = _f1_call(sa, sb, g1a, g1b, dinv, _row(b1), w2)
    s2 = _sc_agg(g2, srcp, dstp)
    g3 = _f2_call(s2, g2, dinv, _row(b2), w3)
    s3 = _sc_agg(g3, srcp, dstp)
    g4 = _f3_call(s3, g3, dinv, _row(b3), w4)
    s4 = _sc_agg(g4, srcp, dstp)
    g5 = _f4_call(s4, g4, dinv, _row(b4), w5)
    s5 = _sc_agg(g5, srcp, dstp)
    h5 = _flast_call(s5, g5, dinv, _row(b5))
    h5 = h5[:_N].reshape(_N // 300, 1200)
    return _f6_call(h5, fc_w, fc_b.reshape(1, 4))


# speedup vs baseline: 44.7625x; 1.1236x over previous
import functools
import math

import jax
import jax.numpy as jnp
from jax import lax
from jax.experimental import pallas as pl
from jax.experimental.pallas import tpu as pltpu
from jax.experimental.pallas import tpu_sc as plsc

_N = 99900
_E = 3196800
_NTILES = 32
_LANES = 16
_GRP = 128
_K = 4
_CHUNK = _K * _GRP
_EPT = _E // _NTILES
_NCHUNK = 198
_EPT_PAD = _NCHUNK * _CHUNK
_ROWTILE = _EPT_PAD // _GRP
_NPAD = 100352
_BLK = 1024
_GRID = _NPAD // _BLK
_BN_SCALE = 1.0 / math.sqrt(1.0 + 1e-5)


def _fill_rows(buf, nrows, val):
    def body(i, c):
        buf[i, :] = jnp.full((_LANES,), val, jnp.float32)
        return c
    lax.fori_loop(0, nrows, body, 0)


def _zero_my_accum_slice(rows, accum, sid, copyrows):
    lo = sid * copyrows
    n_full = copyrows // _CHUNK
    rem = copyrows % _CHUNK
    for z in range(n_full):
        pltpu.sync_copy(rows, accum.at[pl.ds(lo + z * _CHUNK, _CHUNK)])
    if rem:
        pltpu.sync_copy(rows.at[pl.ds(0, rem)],
                        accum.at[pl.ds(lo + n_full * _CHUNK, rem)])


def _make_sc_pass(npad, rowtile, nchunk, with_gather, interpret=False):
    copyrows = npad // 16
    assert nchunk % 3 == 0

    def body(*refs):
        if with_gather:
            (g_hbm, src_hbm, dst_hbm, out_hbm,
             srcb0, srcb1, srcb2, dstb0, dstb1, dstb2,
             rows0, rows1, rows2, accum,
             semi0, semi1, semi2, semg0, semg1, semg2,
             sems0, sems1, sems2) = refs
            srcb = (srcb0, srcb1, srcb2)
            rows = (rows0, rows1, rows2)
            semg = (semg0, semg1, semg2)
        else:
            (dst_hbm, out_hbm, dstb0, dstb1, dstb2, ones, accum,
             semi0, semi1, semi2, sems0, sems1, sems2) = refs
            rows = (ones, ones, ones)
        dstb = (dstb0, dstb1, dstb2)
        semi = (semi0, semi1, semi2)
        sems = (sems0, sems1, sems2)
        cid = lax.axis_index("c")
        sid = lax.axis_index("s")
        wid = cid * 16 + sid
        _fill_rows(rows[0], _CHUNK, 0.0)
        _zero_my_accum_slice(rows[0], accum, sid, copyrows)
        if not with_gather:
            _fill_rows(rows[0], _CHUNK, 1.0)
        plsc.subcore_barrier()
        base = wid * rowtile

        def stage_idx(m, b, sync):
            r0 = base + m * _K
            if sync:
                if with_gather:
                    pltpu.sync_copy(src_hbm.at[pl.ds(r0, _K)], srcb[b])
                pltpu.sync_copy(dst_hbm.at[pl.ds(r0, _K)], dstb[b])
            else:
                if with_gather:
                    pltpu.async_copy(src_hbm.at[pl.ds(r0, _K)], srcb[b],
                                     semi[b])
                pltpu.async_copy(dst_hbm.at[pl.ds(r0, _K)], dstb[b], semi[b])

        def wait_idx(b):
            if with_gather:
                pltpu.make_async_copy(src_hbm.at[pl.ds(base, _K)], srcb[b],
                                      semi[b]).wait()
            pltpu.make_async_copy(dst_hbm.at[pl.ds(base, _K)], dstb[b],
                                  semi[b]).wait()

        def fire_gathers(b):
            for j in range(_K):
                pltpu.async_copy(g_hbm.at[srcb[b].at[j]],
                                 rows[b].at[pl.ds(j * _GRP, _GRP)], semg[b])

        def wait_gathers(b):
            for j in range(_K):
                pltpu.make_async_copy(g_hbm.at[srcb[b].at[j]],
                                      rows[b].at[pl.ds(j * _GRP, _GRP)],
                                      semg[b]).wait()

        def fire_scatters(b):
            for j in range(_K):
                pltpu.async_copy(rows[b].at[pl.ds(j * _GRP, _GRP)],
                                 accum.at[dstb[b].at[j]], sems[b], add=True)

        def wait_scatters(b):
            for j in range(_K):
                pltpu.make_async_copy(rows[b].at[pl.ds(j * _GRP, _GRP)],
                                      accum.at[dstb[b].at[j]], sems[b]).wait()

        for b in range(2):
            stage_idx(b, b, sync=True)
            if with_gather:
                fire_gathers(b)

        def triple(p, c):
            for b in range(3):
                m = 3 * p + b

                @pl.when(m >= 1)
                def _():
                    wait_scatters((b + 2) % 3)

                @pl.when(m + 2 < nchunk)
                def _():
                    stage_idx(m + 2, (b + 2) % 3, sync=False)
                if with_gather:
                    wait_gathers(b)
                fire_scatters(b)

                @pl.when(m + 2 < nchunk)
                def _():
                    wait_idx((b + 2) % 3)
                    if with_gather:
                        fire_gathers((b + 2) % 3)
            return c

        lax.fori_loop(0, nchunk // 3, triple, 0)
        wait_scatters((nchunk - 1) % 3)
        plsc.subcore_barrier()
        lo = sid * copyrows
        pltpu.sync_copy(accum.at[pl.ds(lo, copyrows)],
                        out_hbm.at[cid].at[pl.ds(lo, copyrows)])

    idxbuf = pltpu.VMEM((_K, _GRP), jnp.int32)
    rowbuf = pltpu.VMEM((_CHUNK, _LANES), jnp.float32)
    dma = pltpu.SemaphoreType.DMA
    if with_gather:
        scratch = [idxbuf] * 6 + [rowbuf] * 3 + [
            pltpu.VMEM_SHARED((npad, _LANES), jnp.float32)] + [dma] * 9
    else:
        scratch = [idxbuf] * 3 + [rowbuf] + [
            pltpu.VMEM_SHARED((npad, _LANES), jnp.float32)] + [dma] * 6
    return pl.kernel(
        body,
        out_type=jax.ShapeDtypeStruct((2, npad, _LANES), jnp.float32),
        mesh=plsc.VectorSubcoreMesh(core_axis_name="c", subcore_axis_name="s"),
        scratch_types=scratch,
        compiler_params=pltpu.CompilerParams(use_tc_tiling_on_sc=False),
        interpret=interpret,
    )


def _row_spec(d):
    return pl.BlockSpec((_BLK, d), lambda i: (i, 0))


def _full_spec(shape):
    nd = len(shape)
    return pl.BlockSpec(shape, lambda i, _n=nd: (0,) * _n)


def _part_spec():
    return pl.BlockSpec((2, _BLK, _LANES), lambda i: (0, i, 0))


def _f0_body(p_ref, x_ref, fw1, fb1, g1g, g1b, fw2, fb2, g2g, g2b, w1_ref,
             dinv_ref, ga_ref, gb_ref):
    cnt = p_ref[0, :, 0:1] + p_ref[1, :, 0:1]
    dinv = lax.rsqrt(1.0 + cnt)
    h = x_ref[...] @ fw1[...] + fb1[...]
    h = h * (g1g[...] * _BN_SCALE) + g1b[...]
    h = jnp.maximum(h, 0.0)
    h = h @ fw2[...] + fb2[...]
    h = h * (g2g[...] * _BN_SCALE) + g2b[...]
    g = dinv * (h @ w1_ref[...])
    dinv_ref[...] = dinv
    ga_ref[...] = g[:, :16]
    gb_ref[...] = jnp.concatenate(
        [g[:, 16:], jnp.zeros((_BLK, 7), jnp.float32)], axis=1)


_f0_call = pl.pallas_call(
    _f0_body,
    grid=(_GRID,),
    in_specs=[
        _part_spec(),
        _row_spec(25),
        _full_spec((25, 100)), _full_spec((1, 100)),
        _full_spec((1, 100)), _full_spec((1, 100)),
        _full_spec((100, 25)), _full_spec((1, 25)),
        _full_spec((1, 25)), _full_spec((1, 25)),
        _full_spec((25, 25)),
    ],
    out_specs=[_row_spec(1), _row_spec(16), _row_spec(16)],
    out_shape=[
        jax.ShapeDtypeStruct((_NPAD, 1), jnp.float32),
        jax.ShapeDtypeStruct((_NPAD, 16), jnp.float32),
        jax.ShapeDtypeStruct((_NPAD, 16), jnp.float32),
    ],
)


def _f1_body(sa_ref, sb_ref, ga_ref, gb_ref, dinv_ref, b_ref, w_ref, out_ref):
    agg_a = sa_ref[0] + sa_ref[1] + ga_ref[...]
    agg_b = sb_ref[0] + sb_ref[1] + gb_ref[...]
    agg = jnp.concatenate([agg_a, agg_b[:, :9]], axis=1)
    dinv = dinv_ref[...]
    h = jnp.maximum(dinv * agg + b_ref[...], 0.0)
    out_ref[...] = dinv * (h @ w_ref[...])


_f1_call = pl.pallas_call(
    _f1_body,
    grid=(_GRID,),
    in_specs=[
        _part_spec(), _part_spec(), _row_spec(16), _row_spec(16),
        _row_spec(1), _full_spec((1, 25)), _full_spec((25, 16)),
    ],
    out_specs=_row_spec(16),
    out_shape=jax.ShapeDtypeStruct((_NPAD, 16), jnp.float32),
)


def _make_mid(d, d2):
    def body(s_ref, g_ref, dinv_ref, b_ref, w_ref, out_ref):
        agg = (s_ref[0] + s_ref[1] + g_ref[...])[:, :d]
        dinv = dinv_ref[...]
        h = jnp.maximum(dinv * agg + b_ref[...], 0.0)
        g2 = dinv * (h @ w_ref[...])
        if d2 < 16:
            g2 = jnp.concatenate(
                [g2, jnp.zeros((_BLK, 16 - d2), jnp.float32)], axis=1)
        out_ref[...] = g2

    return pl.pallas_call(
        body,
        grid=(_GRID,),
        in_specs=[
            _part_spec(), _row_spec(16), _row_spec(1),
            _full_spec((1, d)), _full_spec((d, d2)),
        ],
        out_specs=_row_spec(16),
        out_shape=jax.ShapeDtypeStruct((_NPAD, 16), jnp.float32),
    )


_f2_call = _make_mid(16, 16)
_f3_call = _make_mid(16, 8)
_f4_call = _make_mid(8, 4)


def _flast_body(s_ref, g_ref, dinv_ref, b_ref, out_ref):
    agg = (s_ref[0] + s_ref[1] + g_ref[...])[:, :4]
    out_ref[...] = jnp.maximum(dinv_ref[...] * agg + b_ref[...], 0.0)


_flast_call = pl.pallas_call(
    _flast_body,
    grid=(_GRID,),
    in_specs=[_part_spec(), _row_spec(16), _row_spec(1), _full_spec((1, 4))],
    out_specs=_row_spec(4),
    out_shape=jax.ShapeDtypeStruct((_NPAD, 4), jnp.float32),
)


def _f6_body(h_ref, w_ref, b_ref, out_ref):
    out_ref[...] = h_ref[...] @ w_ref[...] + b_ref[...]


_f6_call = pl.pallas_call(
    _f6_body,
    grid=(1,),
    in_specs=[_full_spec((333, 1200)), _full_spec((1200, 4)),
              _full_spec((1, 4))],
    out_specs=_full_spec((333, 4)),
    out_shape=jax.ShapeDtypeStruct((333, 4), jnp.float32),
)


_sc_agg = _make_sc_pass(_NPAD, _ROWTILE, _NCHUNK, with_gather=True)
_sc_deg = _make_sc_pass(_NPAD, _ROWTILE, _NCHUNK, with_gather=False)


def _row(v):
    return v.reshape(1, -1)


def kernel(x, edge_index, ffn_w1, ffn_b1, bn1_g, bn1_b, ffn_w2, ffn_b2,
           bn2_g, bn2_b, w1, b1, w2, b2, w3, b3, w4, b4, w5, b5, fc_w, fc_b):
    f32 = jnp.float32
    x = x.astype(f32)
    src = edge_index[0]
    dst = edge_index[1]
    pad = _EPT_PAD - _EPT
    srcp = jnp.pad(src.reshape(_NTILES, _EPT), ((0, 0), (0, pad)),
                   constant_values=_N).reshape(-1, _GRP)
    dstp = jnp.pad(dst.reshape(_NTILES, _EPT), ((0, 0), (0, pad)),
                   constant_values=_N).reshape(-1, _GRP)
    xp = jnp.pad(x, ((0, _NPAD - _N), (0, 0)))

    p_deg = _sc_deg(dstp)
    dinv, g1a, g1b = _f0_call(
        p_deg, xp, ffn_w1, _row(ffn_b1), _row(bn1_g), _row(bn1_b),
        ffn_w2, _row(ffn_b2), _row(bn2_g), _row(bn2_b), w1)
    sa = _sc_agg(g1a, srcp, dstp)
    sb = _sc_agg(g1b, srcp, dstp)
    g2 = _f1_call(sa, sb, g1a, g1b, dinv, _row(b1), w2)
    s2 = _sc_agg(g2, srcp, dstp)
    g3 = _f2_call(s2, g2, dinv, _row(b2), w3)
    s3 = _sc_agg(g3, srcp, dstp)
    g4 = _f3_call(s3, g3, dinv, _row(b3), w4)
    s4 = _sc_agg(g4, srcp, dstp)
    g5 = _f4_call(s4, g4, dinv, _row(b4), w5)
    s5 = _sc_agg(g5, srcp, dstp)
    h5 = _flast_call(s5, g5, dinv, _row(b5))
    h5 = h5[:_N].reshape(_N // 300, 1200)
    return _f6_call(h5, fc_w, fc_b.reshape(1, 4))


# speedup vs baseline: 44.7895x; 1.0006x over previous
import functools
import math

import jax
import jax.numpy as jnp
from jax import lax
from jax.experimental import pallas as pl
from jax.experimental.pallas import tpu as pltpu
from jax.experimental.pallas import tpu_sc as plsc

_N = 99900
_E = 3196800
_NTILES = 32
_LANES = 16
_GRP = 512
_K = 1
_CHUNK = _K * _GRP
_EPT = _E // _NTILES
_NCHUNK = 198
_EPT_PAD = _NCHUNK * _CHUNK
_ROWTILE = _EPT_PAD // _GRP
_NPAD = 100352
_BLK = 1024
_GRID = _NPAD // _BLK
_BN_SCALE = 1.0 / math.sqrt(1.0 + 1e-5)


def _fill_rows(buf, nrows, val):
    def body(i, c):
        buf[i, :] = jnp.full((_LANES,), val, jnp.float32)
        return c
    lax.fori_loop(0, nrows, body, 0)


def _zero_my_accum_slice(rows, accum, sid, copyrows):
    lo = sid * copyrows
    n_full = copyrows // _CHUNK
    rem = copyrows % _CHUNK
    for z in range(n_full):
        pltpu.sync_copy(rows, accum.at[pl.ds(lo + z * _CHUNK, _CHUNK)])
    if rem:
        pltpu.sync_copy(rows.at[pl.ds(0, rem)],
                        accum.at[pl.ds(lo + n_full * _CHUNK, rem)])


def _make_sc_pass(npad, rowtile, nchunk, with_gather, interpret=False):
    copyrows = npad // 16
    assert nchunk % 3 == 0

    def body(*refs):
        if with_gather:
            (g_hbm, src_hbm, dst_hbm, out_hbm,
             srcb0, srcb1, srcb2, dstb0, dstb1, dstb2,
             rows0, rows1, rows2, accum,
             semi0, semi1, semi2, semg0, semg1, semg2,
             sems0, sems1, sems2) = refs
            srcb = (srcb0, srcb1, srcb2)
            rows = (rows0, rows1, rows2)
            semg = (semg0, semg1, semg2)
        else:
            (dst_hbm, out_hbm, dstb0, dstb1, dstb2, ones, accum,
             semi0, semi1, semi2, sems0, sems1, sems2) = refs
            rows = (ones, ones, ones)
        dstb = (dstb0, dstb1, dstb2)
        semi = (semi0, semi1, semi2)
        sems = (sems0, sems1, sems2)
        cid = lax.axis_index("c")
        sid = lax.axis_index("s")
        wid = cid * 16 + sid
        _fill_rows(rows[0], _CHUNK, 0.0)
        _zero_my_accum_slice(rows[0], accum, sid, copyrows)
        if not with_gather:
            _fill_rows(rows[0], _CHUNK, 1.0)
        plsc.subcore_barrier()
        base = wid * rowtile

        def stage_idx(m, b, sync):
            r0 = base + m * _K
            if sync:
                if with_gather:
                    pltpu.sync_copy(src_hbm.at[pl.ds(r0, _K)], srcb[b])
                pltpu.sync_copy(dst_hbm.at[pl.ds(r0, _K)], dstb[b])
            else:
                if with_gather:
                    pltpu.async_copy(src_hbm.at[pl.ds(r0, _K)], srcb[b],
                                     semi[b])
                pltpu.async_copy(dst_hbm.at[pl.ds(r0, _K)], dstb[b], semi[b])

        def wait_idx(b):
            if with_gather:
                pltpu.make_async_copy(src_hbm.at[pl.ds(base, _K)], srcb[b],
                                      semi[b]).wait()
            pltpu.make_async_copy(dst_hbm.at[pl.ds(base, _K)], dstb[b],
                                  semi[b]).wait()

        def fire_gathers(b):
            for j in range(_K):
                pltpu.async_copy(g_hbm.at[srcb[b].at[j]],
                                 rows[b].at[pl.ds(j * _GRP, _GRP)], semg[b])

        def wait_gathers(b):
            for j in range(_K):
                pltpu.make_async_copy(g_hbm.at[srcb[b].at[j]],
                                      rows[b].at[pl.ds(j * _GRP, _GRP)],
                                      semg[b]).wait()

        def fire_scatters(b):
            for j in range(_K):
                pltpu.async_copy(rows[b].at[pl.ds(j * _GRP, _GRP)],
                                 accum.at[dstb[b].at[j]], sems[b], add=True)

        def wait_scatters(b):
            for j in range(_K):
                pltpu.make_async_copy(rows[b].at[pl.ds(j * _GRP, _GRP)],
                                      accum.at[dstb[b].at[j]], sems[b]).wait()

        for b in range(2):
            stage_idx(b, b, sync=True)
            if with_gather:
                fire_gathers(b)

        def triple(p, c):
            for b in range(3):
                m = 3 * p + b

                @pl.when(m >= 1)
                def _():
                    wait_scatters((b + 2) % 3)

                @pl.when(m + 2 < nchunk)
                def _():
                    stage_idx(m + 2, (b + 2) % 3, sync=False)
                if with_gather:
                    wait_gathers(b)
                fire_scatters(b)

                @pl.when(m + 2 < nchunk)
                def _():
                    wait_idx((b + 2) % 3)
                    if with_gather:
                        fire_gathers((b + 2) % 3)
            return c

        lax.fori_loop(0, nchunk // 3, triple, 0)
        wait_scatters((nchunk - 1) % 3)
        plsc.subcore_barrier()
        lo = sid * copyrows
        pltpu.sync_copy(accum.at[pl.ds(lo, copyrows)],
                        out_hbm.at[cid].at[pl.ds(lo, copyrows)])

    idxbuf = pltpu.VMEM((_K, _GRP), jnp.int32)
    rowbuf = pltpu.VMEM((_CHUNK, _LANES), jnp.float32)
    dma = pltpu.SemaphoreType.DMA
    if with_gather:
        scratch = [idxbuf] * 6 + [rowbuf] * 3 + [
            pltpu.VMEM_SHARED((npad, _LANES), jnp.float32)] + [dma] * 9
    else:
        scratch = [idxbuf] * 3 + [rowbuf] + [
            pltpu.VMEM_SHARED((npad, _LANES), jnp.float32)] + [dma] * 6
    return pl.kernel(
        body,
        out_type=jax.ShapeDtypeStruct((2, npad, _LANES), jnp.float32),
        mesh=plsc.VectorSubcoreMesh(core_axis_name="c", subcore_axis_name="s"),
        scratch_types=scratch,
        compiler_params=pltpu.CompilerParams(use_tc_tiling_on_sc=False),
        interpret=interpret,
    )


def _row_spec(d):
    return pl.BlockSpec((_BLK, d), lambda i: (i, 0))


def _full_spec(shape):
    nd = len(shape)
    return pl.BlockSpec(shape, lambda i, _n=nd: (0,) * _n)


def _part_spec():
    return pl.BlockSpec((2, _BLK, _LANES), lambda i: (0, i, 0))


def _f0_body(p_ref, x_ref, fw1, fb1, g1g, g1b, fw2, fb2, g2g, g2b, w1_ref,
             dinv_ref, ga_ref, gb_ref):
    cnt = p_ref[0, :, 0:1] + p_ref[1, :, 0:1]
    dinv = lax.rsqrt(1.0 + cnt)
    h = x_ref[...] @ fw1[...] + fb1[...]
    h = h * (g1g[...] * _BN_SCALE) + g1b[...]
    h = jnp.maximum(h, 0.0)
    h = h @ fw2[...] + fb2[...]
    h = h * (g2g[...] * _BN_SCALE) + g2b[...]
    g = dinv * (h @ w1_ref[...])
    dinv_ref[...] = dinv
    ga_ref[...] = g[:, :16]
    gb_ref[...] = jnp.concatenate(
        [g[:, 16:], jnp.zeros((_BLK, 7), jnp.float32)], axis=1)


_f0_call = pl.pallas_call(
    _f0_body,
    grid=(_GRID,),
    in_specs=[
        _part_spec(),
        _row_spec(25),
        _full_spec((25, 100)), _full_spec((1, 100)),
        _full_spec((1, 100)), _full_spec((1, 100)),
        _full_spec((100, 25)), _full_spec((1, 25)),
        _full_spec((1, 25)), _full_spec((1, 25)),
        _full_spec((25, 25)),
    ],
    out_specs=[_row_spec(1), _row_spec(16), _row_spec(16)],
    out_shape=[
        jax.ShapeDtypeStruct((_NPAD, 1), jnp.float32),
        jax.ShapeDtypeStruct((_NPAD, 16), jnp.float32),
        jax.ShapeDtypeStruct((_NPAD, 16), jnp.float32),
    ],
)


def _f1_body(sa_ref, sb_ref, ga_ref, gb_ref, dinv_ref, b_ref, w_ref, out_ref):
    agg_a = sa_ref[0] + sa_ref[1] + ga_ref[...]
    agg_b = sb_ref[0] + sb_ref[1] + gb_ref[...]
    agg = jnp.concatenate([agg_a, agg_b[:, :9]], axis=1)
    dinv = dinv_ref[...]
    h = jnp.maximum(dinv * agg + b_ref[...], 0.0)
    out_ref[...] = dinv * (h @ w_ref[...])


_f1_call = pl.pallas_call(
    _f1_body,
    grid=(_GRID,),
    in_specs=[
        _part_spec(), _part_spec(), _row_spec(16), _row_spec(16),
        _row_spec(1), _full_spec((1, 25)), _full_spec((25, 16)),
    ],
    out_specs=_row_spec(16),
    out_shape=jax.ShapeDtypeStruct((_NPAD, 16), jnp.float32),
)


def _make_mid(d, d2):
    def body(s_ref, g_ref, dinv_ref, b_ref, w_ref, out_ref):
        agg = (s_ref[0] + s_ref[1] + g_ref[...])[:, :d]
        dinv = dinv_ref[...]
        h = jnp.maximum(dinv * agg + b_ref[...], 0.0)
        g2 = dinv * (h @ w_ref[...])
        if d2 < 16:
            g2 = jnp.concatenate(
                [g2, jnp.zeros((_BLK, 16 - d2), jnp.float32)], axis=1)
        out_ref[...] = g2

    return pl.pallas_call(
        body,
        grid=(_GRID,),
        in_specs=[
            _part_spec(), _row_spec(16), _row_spec(1),
            _full_spec((1, d)), _full_spec((d, d2)),
        ],
        out_specs=_row_spec(16),
        out_shape=jax.ShapeDtypeStruct((_NPAD, 16), jnp.float32),
    )


_f2_call = _make_mid(16, 16)
_f3_call = _make_mid(16, 8)
_f4_call = _make_mid(8, 4)


def _flast_body(s_ref, g_ref, dinv_ref, b_ref, out_ref):
    agg = (s_ref[0] + s_ref[1] + g_ref[...])[:, :4]
    out_ref[...] = jnp.maximum(dinv_ref[...] * agg + b_ref[...], 0.0)


_flast_call = pl.pallas_call(
    _flast_body,
    grid=(_GRID,),
    in_specs=[_part_spec(), _row_spec(16), _row_spec(1), _full_spec((1, 4))],
    out_specs=_row_spec(4),
    out_shape=jax.ShapeDtypeStruct((_NPAD, 4), jnp.float32),
)


def _f6_body(h_ref, w_ref, b_ref, out_ref):
    out_ref[...] = h_ref[...] @ w_ref[...] + b_ref[...]


_f6_call = pl.pallas_call(
    _f6_body,
    grid=(1,),
    in_specs=[_full_spec((333, 1200)), _full_spec((1200, 4)),
              _full_spec((1, 4))],
    out_specs=_full_spec((333, 4)),
    out_shape=jax.ShapeDtypeStruct((333, 4), jnp.float32),
)


_sc_agg = _make_sc_pass(_NPAD, _ROWTILE, _NCHUNK, with_gather=True)
_sc_deg = _make_sc_pass(_NPAD, _ROWTILE, _NCHUNK, with_gather=False)


def _row(v):
    return v.reshape(1, -1)


def kernel(x, edge_index, ffn_w1, ffn_b1, bn1_g, bn1_b, ffn_w2, ffn_b2,
           bn2_g, bn2_b, w1, b1, w2, b2, w3, b3, w4, b4, w5, b5, fc_w, fc_b):
    f32 = jnp.float32
    x = x.astype(f32)
    src = edge_index[0]
    dst = edge_index[1]
    pad = _EPT_PAD - _EPT
    srcp = jnp.pad(src.reshape(_NTILES, _EPT), ((0, 0), (0, pad)),
                   constant_values=_N).reshape(-1, _GRP)
    dstp = jnp.pad(dst.reshape(_NTILES, _EPT), ((0, 0), (0, pad)),
                   constant_values=_N).reshape(-1, _GRP)
    xp = jnp.pad(x, ((0, _NPAD - _N), (0, 0)))

    p_deg = _sc_deg(dstp)
    dinv, g1a, g1b = _f0_call(
        p_deg, xp, ffn_w1, _row(ffn_b1), _row(bn1_g), _row(bn1_b),
        ffn_w2, _row(ffn_b2), _row(bn2_g), _row(bn2_b), w1)
    sa = _sc_agg(g1a, srcp, dstp)
    sb = _sc_agg(g1b, srcp, dstp)
    g2 = _f1_call(sa, sb, g1a, g1b, dinv, _row(b1), w2)
    s2 = _sc_agg(g2, srcp, dstp)
    g3 = _f2_call(s2, g2, dinv, _row(b2), w3)
    s3 = _sc_agg(g3, srcp, dstp)
    g4 = _f3_call(s3, g3, dinv, _row(b3), w4)
    s4 = _sc_agg(g4, srcp, dstp)
    g5 = _f4_call(s4, g4, dinv, _row(b4), w5)
    s5 = _sc_agg(g5, srcp, dstp)
    h5 = _flast_call(s5, g5, dinv, _row(b5))
    h5 = h5[:_N].reshape(_N // 300, 1200)
    return _f6_call(h5, fc_w, fc_b.reshape(1, 4))


# speedup vs baseline: 52.8463x; 1.1799x over previous
import math

import jax
import jax.numpy as jnp
from jax import lax
from jax.experimental import pallas as pl
from jax.experimental.pallas import tpu as pltpu
from jax.experimental.pallas import tpu_sc as plsc

_N = 99900
_E = 3196800
_NTILES = 32
_LANES = 16
_GRP = 512
_K = 1
_CHUNK = _K * _GRP
_NCHUNK = 198
_EPT_PAD = _NCHUNK * _CHUNK
_ROWTILE = _EPT_PAD // _GRP
_NPAD = 100352
_PROW = _NPAD * _LANES // 128
_PBLK = 128
_GRID = _PROW // _PBLK
_BN_SCALE = 1.0 / math.sqrt(1.0 + 1e-5)


def _fill_rows(buf, nrows, val):
    def body(i, c):
        buf[i, :] = jnp.full((_LANES,), val, jnp.float32)
        return c
    lax.fori_loop(0, nrows, body, 0)


def _zero_my_accum_slice(rows, accum, sid, copyrows):
    lo = sid * copyrows
    n_full = copyrows // _CHUNK
    rem = copyrows % _CHUNK
    for z in range(n_full):
        pltpu.sync_copy(rows, accum.at[pl.ds(lo + z * _CHUNK, _CHUNK)])
    if rem:
        pltpu.sync_copy(rows.at[pl.ds(0, rem)],
                        accum.at[pl.ds(lo + n_full * _CHUNK, rem)])


def _make_sc_pass(npad, rowtile, nchunk, with_gather, interpret=False):
    copyrows = npad // 16
    assert nchunk % 3 == 0

    def body(*refs):
        if with_gather:
            (gp_hbm, src_hbm, dst_hbm, out0, out1,
             srcb0, srcb1, srcb2, dstb0, dstb1, dstb2,
             rows0, rows1, rows2, accum,
             semi0, semi1, semi2, semg0, semg1, semg2,
             sems0, sems1, sems2) = refs
            g_hbm = gp_hbm
            srcb = (srcb0, srcb1, srcb2)
            rows = (rows0, rows1, rows2)
            semg = (semg0, semg1, semg2)
        else:
            (dst_hbm, out0, out1, dstb0, dstb1, dstb2, ones, accum,
             semi0, semi1, semi2, sems0, sems1, sems2) = refs
            rows = (ones, ones, ones)
        dstb = (dstb0, dstb1, dstb2)
        semi = (semi0, semi1, semi2)
        sems = (sems0, sems1, sems2)
        cid = lax.axis_index("c")
        sid = lax.axis_index("s")
        wid = cid * 16 + sid
        _fill_rows(rows[0], _CHUNK, 0.0)
        _zero_my_accum_slice(rows[0], accum, sid, copyrows)
        if not with_gather:
            _fill_rows(rows[0], _CHUNK, 1.0)
        plsc.subcore_barrier()
        base = wid * rowtile

        def stage_idx(m, b, sync):
            r0 = base + m * _K
            if sync:
                if with_gather:
                    pltpu.sync_copy(src_hbm.at[pl.ds(r0, _K)], srcb[b])
                pltpu.sync_copy(dst_hbm.at[pl.ds(r0, _K)], dstb[b])
            else:
                if with_gather:
                    pltpu.async_copy(src_hbm.at[pl.ds(r0, _K)], srcb[b],
                                     semi[b])
                pltpu.async_copy(dst_hbm.at[pl.ds(r0, _K)], dstb[b], semi[b])

        def wait_idx(b):
            if with_gather:
                pltpu.make_async_copy(src_hbm.at[pl.ds(base, _K)], srcb[b],
                                      semi[b]).wait()
            pltpu.make_async_copy(dst_hbm.at[pl.ds(base, _K)], dstb[b],
                                  semi[b]).wait()

        def fire_gathers(b):
            for j in range(_K):
                pltpu.async_copy(g_hbm.at[srcb[b].at[j]],
                                 rows[b].at[pl.ds(j * _GRP, _GRP)], semg[b])

        def wait_gathers(b):
            for j in range(_K):
                pltpu.make_async_copy(g_hbm.at[srcb[b].at[j]],
                                      rows[b].at[pl.ds(j * _GRP, _GRP)],
                                      semg[b]).wait()

        def fire_scatters(b):
            for j in range(_K):
                pltpu.async_copy(rows[b].at[pl.ds(j * _GRP, _GRP)],
                                 accum.at[dstb[b].at[j]], sems[b], add=True)

        def wait_scatters(b):
            for j in range(_K):
                pltpu.make_async_copy(rows[b].at[pl.ds(j * _GRP, _GRP)],
                                      accum.at[dstb[b].at[j]], sems[b]).wait()

        for b in range(2):
            stage_idx(b, b, sync=True)
            if with_gather:
                fire_gathers(b)

        def triple(p, c):
            for b in range(3):
                m = 3 * p + b

                @pl.when(m >= 1)
                def _():
                    wait_scatters((b + 2) % 3)

                @pl.when(m + 2 < nchunk)
                def _():
                    stage_idx(m + 2, (b + 2) % 3, sync=False)
                if with_gather:
                    wait_gathers(b)
                fire_scatters(b)

                @pl.when(m + 2 < nchunk)
                def _():
                    wait_idx((b + 2) % 3)
                    if with_gather:
                        fire_gathers((b + 2) % 3)
            return c

        lax.fori_loop(0, nchunk // 3, triple, 0)
        wait_scatters((nchunk - 1) % 3)
        plsc.subcore_barrier()
        lo = sid * copyrows

        @pl.when(cid == 0)
        def _():
            pltpu.sync_copy(accum.at[pl.ds(lo, copyrows)],
                            out0.at[pl.ds(lo, copyrows)])

        @pl.when(cid == 1)
        def _():
            pltpu.sync_copy(accum.at[pl.ds(lo, copyrows)],
                            out1.at[pl.ds(lo, copyrows)])

    idxbuf = pltpu.VMEM((_K, _GRP), jnp.int32)
    rowbuf = pltpu.VMEM((_CHUNK, _LANES), jnp.float32)
    dma = pltpu.SemaphoreType.DMA
    part = jax.ShapeDtypeStruct((npad, _LANES), jnp.float32)
    if with_gather:
        scratch = [idxbuf] * 6 + [rowbuf] * 3 + [
            pltpu.VMEM_SHARED((npad, _LANES), jnp.float32)] + [dma] * 9
    else:
        scratch = [idxbuf] * 3 + [rowbuf] + [
            pltpu.VMEM_SHARED((npad, _LANES), jnp.float32)] + [dma] * 6
    return pl.kernel(
        body,
        out_type=(part, part),
        mesh=plsc.VectorSubcoreMesh(core_axis_name="c", subcore_axis_name="s"),
        scratch_types=scratch,
        compiler_params=pltpu.CompilerParams(use_tc_tiling_on_sc=False),
        interpret=interpret,
    )


_sc_agg_raw = _make_sc_pass(_NPAD, _ROWTILE, _NCHUNK, with_gather=True)
_sc_deg_raw = _make_sc_pass(_NPAD, _ROWTILE, _NCHUNK, with_gather=False)


def _sc_agg(gp, srcp, dstp):
    o0, o1 = _sc_agg_raw(gp.reshape(_NPAD, _LANES), srcp, dstp)
    return o0.reshape(_PROW, 128), o1.reshape(_PROW, 128)


def _sc_deg(dstp):
    o0, o1 = _sc_deg_raw(dstp)
    return o0.reshape(_PROW, 128), o1.reshape(_PROW, 128)


def _pspec():
    return pl.BlockSpec((_PBLK, 128), lambda i: (i, 0))


def _full_spec(shape):
    nd = len(shape)
    return pl.BlockSpec(shape, lambda i, _n=nd: (0,) * _n)


def _f0_body(p0_ref, p1_ref, xa_ref, xb_ref,
             w1f_ref, sh1_ref, w2f_ref, sh2_ref, w1_ref,
             dinv_ref, ga_ref, gb_ref):
    dinv = lax.rsqrt(1.0 + p0_ref[...] + p1_ref[...])
    xcat = jnp.concatenate([xa_ref[...], xb_ref[...]], axis=1)
    h1 = jnp.concatenate(
        [jnp.maximum(xcat @ w1f_ref[k] + sh1_ref[k], 0.0) for k in range(7)],
        axis=1)
    h2a = h1 @ w2f_ref[0] + sh2_ref[0]
    h2b = h1 @ w2f_ref[1] + sh2_ref[1]
    hcat = jnp.concatenate([h2a, h2b], axis=1)
    dinv_ref[...] = dinv
    ga_ref[...] = dinv * (hcat @ w1_ref[0])
    gb_ref[...] = dinv * (hcat @ w1_ref[1])


_f0_call = pl.pallas_call(
    _f0_body,
    grid=(_GRID,),
    in_specs=[
        _pspec(), _pspec(), _pspec(), _pspec(),
        _full_spec((7, 256, 128)), _full_spec((7, 1, 128)),
        _full_spec((2, 896, 128)), _full_spec((2, 1, 128)),
        _full_spec((2, 256, 128)),
    ],
    out_specs=[_pspec(), _pspec(), _pspec()],
    out_shape=[jax.ShapeDtypeStruct((_PROW, 128), jnp.float32)] * 3,
)


def _f1_body(sa0, sa1, sb0, sb1, ga_ref, gb_ref, dinv_ref,
             bta_ref, btb_ref, w2_ref, out_ref):
    dinv = dinv_ref[...]
    ha = jnp.maximum(dinv * (sa0[...] + sa1[...] + ga_ref[...])
                     + bta_ref[...], 0.0)
    hb = jnp.maximum(dinv * (sb0[...] + sb1[...] + gb_ref[...])
                     + btb_ref[...], 0.0)
    hcat = jnp.concatenate([ha, hb], axis=1)
    out_ref[...] = dinv * (hcat @ w2_ref[...])


_f1_call = pl.pallas_call(
    _f1_body,
    grid=(_GRID,),
    in_specs=[_pspec()] * 7 + [
        _full_spec((1, 128)), _full_spec((1, 128)), _full_spec((256, 128))],
    out_specs=_pspec(),
    out_shape=jax.ShapeDtypeStruct((_PROW, 128), jnp.float32),
)


def _fmid_body(s0, s1, g_ref, dinv_ref, bt_ref, w_ref, out_ref):
    dinv = dinv_ref[...]
    h = jnp.maximum(dinv * (s0[...] + s1[...] + g_ref[...]) + bt_ref[...],
                    0.0)
    out_ref[...] = dinv * (h @ w_ref[...])


_fmid_call = pl.pallas_call(
    _fmid_body,
    grid=(_GRID,),
    in_specs=[_pspec()] * 4 + [_full_spec((1, 128)), _full_spec((128, 128))],
    out_specs=_pspec(),
    out_shape=jax.ShapeDtypeStruct((_PROW, 128), jnp.float32),
)


def _flast_body(s0, s1, g_ref, dinv_ref, bt_ref, out_ref):
    h = jnp.maximum(
        dinv_ref[...] * (s0[...] + s1[...] + g_ref[...]) + bt_ref[...], 0.0)
    out_ref[...] = h


_flast_call = pl.pallas_call(
    _flast_body,
    grid=(_GRID,),
    in_specs=[_pspec()] * 4 + [_full_spec((1, 128))],
    out_specs=_pspec(),
    out_shape=jax.ShapeDtypeStruct((_PROW, 128), jnp.float32),
)


def _f6_body(h_ref, w_ref, b_ref, out_ref):
    out_ref[...] = h_ref[...] @ w_ref[...] + b_ref[...]


_f6_call = pl.pallas_call(
    _f6_body,
    grid=(1,),
    in_specs=[_full_spec((333, 1200)), _full_spec((1200, 4)),
              _full_spec((1, 4))],
    out_specs=_full_spec((333, 4)),
    out_shape=jax.ShapeDtypeStruct((333, 4), jnp.float32),
)


def _bd(w16):
    return jnp.kron(jnp.eye(8, dtype=jnp.float32), w16)


def _pad2(w, r, c):
    return jnp.pad(w, ((0, r - w.shape[0]), (0, c - w.shape[1])))


def _tile8(v16):
    return jnp.tile(v16, 8).reshape(1, 128)


def kernel(x, edge_index, ffn_w1, ffn_b1, bn1_g, bn1_b, ffn_w2, ffn_b2,
           bn2_g, bn2_b, w1, b1, w2, b2, w3, b3, w4, b4, w5, b5, fc_w, fc_b):
    f32 = jnp.float32
    x = x.astype(f32)
    pad = _NTILES * _EPT_PAD - _E
    srcp = jnp.pad(edge_index[0], (0, pad),
                   constant_values=_N).reshape(-1, _GRP)
    dstp = jnp.pad(edge_index[1], (0, pad),
                   constant_values=_N).reshape(-1, _GRP)
    xq = jnp.pad(x, ((0, _NPAD - _N), (0, 7)))
    xa = xq[:, :16].reshape(_PROW, 128)
    xb = xq[:, 16:].reshape(_PROW, 128)

    s1 = bn1_g * _BN_SCALE
    w1f = ffn_w1 * s1[None, :]
    sh1 = ffn_b1 * s1 + bn1_b
    s2 = bn2_g * _BN_SCALE
    w2f = ffn_w2 * s2[None, :]
    sh2 = ffn_b2 * s2 + bn2_b
    w1fp = _pad2(w1f, 32, 112)
    w2fp = _pad2(w2f, 112, 32)
    w1p = _pad2(w1, 32, 32)
    sh1p = jnp.pad(sh1, (0, 12))
    sh2p = jnp.pad(sh2, (0, 7))
    w1f_bd = jnp.stack([
        jnp.concatenate([_bd(w1fp[0:16, 16 * k:16 * k + 16]),
                         _bd(w1fp[16:32, 16 * k:16 * k + 16])], axis=0)
        for k in range(7)])
    sh1t = jnp.stack([_tile8(sh1p[16 * k:16 * k + 16]) for k in range(7)])
    w2f_bd = jnp.stack([
        jnp.concatenate([_bd(w2fp[16 * k:16 * k + 16, 16 * c:16 * c + 16])
                         for k in range(7)], axis=0)
        for c in range(2)])
    sh2t = jnp.stack([_tile8(sh2p[0:16]), _tile8(sh2p[16:32])])
    w1_bd = jnp.stack([
        jnp.concatenate([_bd(w1p[0:16, 16 * c:16 * c + 16]),
                         _bd(w1p[16:32, 16 * c:16 * c + 16])], axis=0)
        for c in range(2)])
    b1p = jnp.pad(b1, (0, 7))
    bt1a = _tile8(b1p[0:16])
    bt1b = _tile8(b1p[16:32])
    w2p = _pad2(w2, 32, 16)
    w2_st = jnp.concatenate([_bd(w2p[0:16]), _bd(w2p[16:32])], axis=0)
    bd3 = _bd(w3)
    bd4 = _bd(_pad2(w4, 16, 16))
    bd5 = _bd(_pad2(w5, 16, 16))
    bt2 = _tile8(b2)
    bt3 = _tile8(b3)
    bt4 = _tile8(jnp.pad(b4, (0, 8)))
    bt5 = _tile8(jnp.pad(b5, (0, 12)))

    p0, p1 = _sc_deg(dstp)
    dinv, g1a, g1b = _f0_call(p0, p1, xa, xb, w1f_bd, sh1t, w2f_bd, sh2t,
                              w1_bd)
    sa0, sa1 = _sc_agg(g1a, srcp, dstp)
    sb0, sb1 = _sc_agg(g1b, srcp, dstp)
    g2 = _f1_call(sa0, sa1, sb0, sb1, g1a, g1b, dinv, bt1a, bt1b, w2_st)
    s0, s1_ = _sc_agg(g2, srcp, dstp)
    g3 = _fmid_call(s0, s1_, g2, dinv, bt2, bd3)
    s0, s1_ = _sc_agg(g3, srcp, dstp)
    g4 = _fmid_call(s0, s1_, g3, dinv, bt3, bd4)
    s0, s1_ = _sc_agg(g4, srcp, dstp)
    g5 = _fmid_call(s0, s1_, g4, dinv, bt4, bd5)
    s0, s1_ = _sc_agg(g5, srcp, dstp)
    h5 = _flast_call(s0, s1_, g5, dinv, bt5)
    h5 = h5.reshape(_NPAD, 16)[:_N, :4].reshape(_N // 300, 1200)
    return _f6_call(h5, fc_w, fc_b.reshape(1, 4))
